# baseline (speedup 1.0000x reference)
import jax
import jax.numpy as jnp
from jax.experimental import pallas as pl
from jax.experimental.pallas import tpu as pltpu

N_LEADS = 12
D_MODEL = 256
BATCH = 16384
BLOCK_B = 512


def _body(pos_ref, w_ref, x_ref, o_ref):
    pos = pos_ref[...]
    cols = jax.lax.broadcasted_iota(jnp.int32, (N_LEADS, N_LEADS), 1)
    onehot = (pos.reshape(N_LEADS, 1) == cols).astype(jnp.float32)
    pos_enc = jnp.dot(onehot, w_ref[...], preferred_element_type=jnp.float32,
                      precision=jax.lax.Precision.HIGHEST)
    o_ref[...] = x_ref[...] + pos_enc[None, :, :]


def kernel(x, encoding_weight, positions):
    pos2d = positions.astype(jnp.int32).reshape(1, N_LEADS)
    grid = (BATCH // BLOCK_B,)
    return pl.pallas_call(
        _body,
        grid=grid,
        in_specs=[
            pl.BlockSpec((1, N_LEADS), lambda i: (0, 0)),
            pl.BlockSpec((N_LEADS, D_MODEL), lambda i: (0, 0)),
            pl.BlockSpec((BLOCK_B, N_LEADS, D_MODEL), lambda i: (i, 0, 0)),
        ],
        out_specs=pl.BlockSpec((BLOCK_B, N_LEADS, D_MODEL), lambda i: (i, 0, 0)),
        out_shape=jax.ShapeDtypeStruct((BATCH, N_LEADS, D_MODEL), jnp.float32),
    )(pos2d, encoding_weight, x)
